# baseline (speedup 1.0000x reference)
import functools

import jax
import jax.numpy as jnp
from jax import lax
from jax.experimental import pallas as pl
from jax.experimental.pallas import tpu as pltpu
from jax.experimental.pallas import tpu_sc as plsc

TH_CONF = 0.05
TH_IOU = 0.5
MAX_DET = 100
NEG = -1e9
B, N, C = 4, 20000, 21
NFG = C - 1
NROW = B * NFG
KPAD = 128
BIGI = 2**30

K1BASE = 0x3D4C0000 >> 12
NB1 = (0x3F800000 >> 12) - K1BASE + 1
NB1P = ((NB1 + 15) // 16) * 16
NB2 = 4096
NCH = N // 16


def _prep_body(conf_ref, loc_ref, anch_ref, s_ref, bx_ref):
    c = conf_ref[0]
    m = jnp.max(c, axis=0, keepdims=True)
    e = jnp.exp(c - m)
    den = jnp.sum(e, axis=0, keepdims=True)
    s = e[1:, :] / den
    s_ref[0] = jnp.where(s >= TH_CONF, s, NEG)

    l = loc_ref[0]
    a = anch_ref[...]
    acx, acy, aw, ah = a[0:1], a[1:2], a[2:3], a[3:4]
    cx = acx + l[0:1] * 0.1 * aw
    cy = acy + l[1:2] * 0.1 * ah
    w = aw * jnp.exp(l[2:3] * 0.2)
    h = ah * jnp.exp(l[3:4] * 0.2)
    bx_ref[0] = jnp.concatenate(
        [cx - w * 0.5, cy - h * 0.5, cx + w * 0.5, cy + h * 0.5], axis=0)


def _stage1(conf_t, loc_t, anch_t):
    return pl.pallas_call(
        _prep_body,
        grid=(B,),
        in_specs=[
            pl.BlockSpec((1, C, N), lambda b: (b, 0, 0)),
            pl.BlockSpec((1, 4, N), lambda b: (b, 0, 0)),
            pl.BlockSpec((4, N), lambda b: (0, 0)),
        ],
        out_specs=[
            pl.BlockSpec((1, NFG, N), lambda b: (b, 0, 0)),
            pl.BlockSpec((1, 4, N), lambda b: (b, 0, 0)),
        ],
        out_shape=[
            jax.ShapeDtypeStruct((B, NFG, N), jnp.float32),
            jax.ShapeDtypeStruct((B, 4, N), jnp.float32),
        ],
    )(conf_t, loc_t, anch_t)


def _hist_walk(hist_ref, nblk, base_count, lane_iota):
    def cond_f(carry):
        i, found, cum, cut, m = carry
        return jnp.logical_and(i < nblk, found == 0)

    def body_f(carry):
        i, found, cum, cut, m = carry
        blk = nblk - 1 - i
        h = hist_ref[pl.ds(blk * 16, 16)]
        tot = jnp.sum(h)
        crossing = (base_count + cum + tot) >= MAX_DET

        def hit_f(_):
            rh = lax.rev(h, (0,))
            cs = jnp.cumsum(rh)
            cross = (base_count + cum + cs) >= MAX_DET
            lstar = jnp.min(jnp.where(cross, lane_iota, 16))
            c_at = jnp.sum(jnp.where(lane_iota == lstar, cs, 0))
            h_at = jnp.sum(jnp.where(lane_iota == lstar, rh, 0))
            return (jnp.int32(1), blk * 16 + 15 - lstar,
                    base_count + cum + c_at - h_at)

        def miss_f(_):
            return (jnp.int32(0), cut, m)

        found, cut, m = lax.cond(crossing, hit_f, miss_f, 0)
        return (i + 1, found, cum + tot, cut, m)

    _, found, cum, cut, m = lax.while_loop(
        cond_f, body_f,
        (jnp.int32(0), jnp.int32(0), jnp.int32(0), jnp.int32(0), jnp.int32(0)))
    return found, cum, cut, m


def _sc_row(r, scores_hbm, boxflat_hbm, so_ref, bo_ref, srow, h1, h2,
            idxb, scb, gidx, grow, cmax, sem):
    lane_iota = lax.iota(jnp.int32, 16)
    pltpu.sync_copy(scores_hbm.at[r], srow)

    zero_v = jnp.zeros((16,), jnp.int32)
    ones_v = jnp.ones((16,), jnp.int32)

    def zero1(i, _):
        for u in range(5):
            h1[pl.ds(i * 80 + u * 16, 16)] = zero_v
        return 0

    def maxpass(j, _):
        gmax = jnp.full((16,), NEG, jnp.float32)
        for u in range(5):
            gmax = jnp.maximum(gmax, srow[pl.ds(j * 80 + u * 16, 16)])
        cmax[pl.ds(j * 16, 16)] = gmax
        return 0
    lax.fori_loop(0, NCH // 5, maxpass, 0)

    lax.fori_loop(0, NB1P // 80, zero1, 0)

    def cmhist(i, _):
        cm = cmax[pl.ds(i * 16, 16)]
        bits = lax.bitcast_convert_type(cm, jnp.int32)
        key = jnp.clip((bits >> 12) - K1BASE, 0, NB1P - 1)
        plsc.addupdate_scatter(h1, [key], ones_v, mask=cm >= TH_CONF)
        return 0
    lax.fori_loop(0, NCH // 5, cmhist, 0)

    found0, _, cut0, _ = _hist_walk(h1, NB1P // 16, jnp.int32(0), lane_iota)
    tau0 = jnp.maximum(jnp.where(
        found0 == 1,
        lax.bitcast_convert_type(
            jnp.broadcast_to((cut0 + K1BASE) << 12, (16,)).astype(jnp.int32),
            jnp.float32),
        jnp.float32(NEG)), jnp.float32(TH_CONF))

    lax.fori_loop(0, NB1P // 80, zero1, 0)

    def hpass1(j, _):
        cm = cmax[pl.ds(j * 16, 16)]
        hitg = jnp.sum((cm >= tau0).astype(jnp.int32)) > 0

        @pl.when(hitg)
        def _():
            for u in range(5):
                v = srow[pl.ds(j * 80 + u * 16, 16)]
                msk = v >= tau0
                bits = lax.bitcast_convert_type(v, jnp.int32)
                key = jnp.clip((bits >> 12) - K1BASE, 0, NB1P - 1)
                plsc.addupdate_scatter(h1, [key], ones_v, mask=msk)
        return 0
    lax.fori_loop(0, NCH // 5, hpass1, 0)

    found1, _, cut1, m1 = _hist_walk(h1, NB1P // 16, jnp.int32(0), lane_iota)

    def zero2(i, _):
        for u in range(4):
            h2[pl.ds(i * 64 + u * 16, 16)] = zero_v
        return 0
    lax.fori_loop(0, NB2 // 64, zero2, 0)

    blo = lax.bitcast_convert_type(
        jnp.broadcast_to((cut1 + K1BASE) << 12, (16,)).astype(jnp.int32),
        jnp.float32)

    def hpass2(j, _):
        cm = cmax[pl.ds(j * 16, 16)]
        hitg = jnp.sum((cm >= blo).astype(jnp.int32)) > 0

        @pl.when(hitg)
        def _():
            for u in range(5):
                v = srow[pl.ds(j * 80 + u * 16, 16)]
                bits = lax.bitcast_convert_type(v, jnp.int32)
                msk = jnp.logical_and(v >= TH_CONF,
                                      ((bits >> 12) - K1BASE) == cut1)
                key = bits & 0xFFF
                plsc.addupdate_scatter(h2, [key], ones_v, mask=msk)
        return 0
    lax.fori_loop(0, NCH // 5, hpass2, 0)

    _, _, cut2, m = _hist_walk(h2, NB2 // 16, m1, lane_iota)

    tau_bits = ((cut1 + K1BASE) << 12) | cut2
    tau_vec = lax.bitcast_convert_type(
        jnp.broadcast_to(tau_bits, (16,)).astype(jnp.int32), jnp.float32)
    tau = jnp.where(found1 == 1, tau_vec, jnp.float32(0.0))
    t_tie = jnp.where(found1 == 1, MAX_DET - m, 0)

    def fill(i, _):
        scb[pl.ds(i * 16, 16)] = jnp.full((16,), NEG, jnp.float32)
        idxb[pl.ds(i * 16, 16)] = jnp.zeros((16,), jnp.int32)
        return 0
    lax.fori_loop(0, KPAD // 16, fill, 0)

    def selstep(j, carry):
        off, ties = carry
        cm = cmax[pl.ds(j * 16, 16)]
        hitg = jnp.sum((cm >= tau).astype(jnp.int32)) > 0

        def do_group(c):
            off, ties = c
            for u in range(5):
                v = srow[pl.ds(j * 80 + u * 16, 16)]
                gt = v > tau
                eq = v == tau
                eqc = jnp.cumsum(eq.astype(jnp.int32))
                take_eq = jnp.logical_and(eq, (ties + eqc) <= t_tie)
                sel = jnp.logical_or(gt, take_eq)
                iv = lane_iota + (j * 80 + u * 16)
                plsc.store_compressed(idxb.at[pl.ds(off, 16)], iv, mask=sel)
                plsc.store_compressed(scb.at[pl.ds(off, 16)], v, mask=sel)
                off = off + jnp.sum(sel.astype(jnp.int32))
                ties = ties + jnp.sum(eq.astype(jnp.int32))
            return off, ties

        return lax.cond(hitg, do_group, lambda c: c, (off, ties))

    lax.fori_loop(0, NCH // 5, selstep, (jnp.int32(0), jnp.int32(0)))

    pltpu.sync_copy(scb, so_ref.at[r])

    bq = r // NFG
    for d in range(4):
        def gi(i, _):
            gidx[pl.ds(d * KPAD + i * 16, 16)] = (
                idxb[pl.ds(i * 16, 16)] + (bq * 4 + d) * N)
            return 0
        lax.fori_loop(0, KPAD // 16, gi, 0)
    copies = [
        pltpu.async_copy(boxflat_hbm.at[gidx.at[pl.ds(d * KPAD, KPAD)]],
                         grow.at[pl.ds(d * KPAD, KPAD)], sem)
        for d in range(4)
    ]
    for c in copies:
        c.wait()
    for d in range(4):
        pltpu.sync_copy(grow.at[pl.ds(d * KPAD, KPAD)], bo_ref.at[r, d])


def _sc_body(scores_hbm, boxflat_hbm, so_ref, bo_ref, srow, h1, h2,
             idxb, scb, gidx, grow, cmax, sem):
    wid = lax.axis_index("s") * 2 + lax.axis_index("c")
    for i in range(3):
        r = wid + 32 * i
        @pl.when(r < NROW)
        def _():
            _sc_row(r, scores_hbm, boxflat_hbm, so_ref, bo_ref, srow, h1, h2,
                    idxb, scb, gidx, grow, cmax, sem)


def _stage2(scores, boxflat):
    mesh = plsc.VectorSubcoreMesh(core_axis_name="c", subcore_axis_name="s")
    f = pl.kernel(
        _sc_body,
        mesh=mesh,
        compiler_params=pltpu.CompilerParams(needs_layout_passes=False),
        out_type=[
            jax.ShapeDtypeStruct((NROW, KPAD), jnp.float32),
            jax.ShapeDtypeStruct((NROW, 4, KPAD), jnp.float32),
        ],
        scratch_types=[
            pltpu.VMEM((N,), jnp.float32),
            pltpu.VMEM((NB1P,), jnp.int32),
            pltpu.VMEM((NB2,), jnp.int32),
            pltpu.VMEM((KPAD,), jnp.int32),
            pltpu.VMEM((KPAD,), jnp.float32),
            pltpu.VMEM((4 * KPAD,), jnp.int32),
            pltpu.VMEM((4 * KPAD,), jnp.float32),
            pltpu.VMEM((4096,), jnp.float32),
            pltpu.SemaphoreType.DMA,
        ],
    )
    return f(scores, boxflat)


def _nms_body(cs_ref, cb_ref, out_ref, s_scr):
    s_scr[...] = cs_ref[...]

    def nms_step(t, _):
        cs = s_scr[...]
        mx = jnp.max(cs, axis=1, keepdims=True)
        iota = lax.broadcasted_iota(jnp.int32, (NROW, KPAD), 1)
        idx = jnp.min(jnp.where(cs == mx, iota, BIGI), axis=1, keepdims=True)
        onehot = iota == idx
        valid = mx > NEG / 2
        cb = cb_ref[...]
        bb = jnp.sum(jnp.where(onehot[None], cb, 0.0), axis=2)
        vrow = valid[:, 0][None, :]
        ob = jnp.where(vrow, bb, 0.0)
        osc = jnp.where(vrow, mx[:, 0][None, :], 0.0)
        val = jnp.concatenate([ob, osc], axis=0)
        slot = iota == t
        out_ref[...] = jnp.where(slot[None], val[:, :, None], out_ref[...])

        bx1, by1 = bb[0][:, None], bb[1][:, None]
        bx2, by2 = bb[2][:, None], bb[3][:, None]
        x1 = jnp.maximum(bx1, cb[0])
        y1 = jnp.maximum(by1, cb[1])
        x2 = jnp.minimum(bx2, cb[2])
        y2 = jnp.minimum(by2, cb[3])
        inter = jnp.maximum(x2 - x1, 0.0) * jnp.maximum(y2 - y1, 0.0)
        area_a = (bx2 - bx1) * (by2 - by1)
        area_b = (cb[2] - cb[0]) * (cb[3] - cb[1])
        iou = inter / (area_a + area_b - inter + 1e-9)
        supp = ((iou > TH_IOU) & valid) | onehot
        s_scr[...] = jnp.where(supp, NEG, cs)
        return 0

    lax.fori_loop(0, MAX_DET, nms_step, 0)


def _stage3(cs, cb):
    return pl.pallas_call(
        _nms_body,
        out_shape=jax.ShapeDtypeStruct((5, NROW, KPAD), jnp.float32),
        scratch_shapes=[pltpu.VMEM((NROW, KPAD), jnp.float32)],
    )(cs, cb)


def kernel(conf, loc, anchors):
    conf_t = jnp.transpose(conf, (0, 2, 1))
    loc_t = jnp.transpose(loc, (0, 2, 1))
    anch_t = jnp.transpose(anchors)

    scores, boxes = _stage1(conf_t, loc_t, anch_t)
    cand_s, cand_b = _stage2(scores.reshape(NROW, N), boxes.reshape(B * 4 * N))
    cs = cand_s
    cb = jnp.transpose(cand_b, (1, 0, 2))
    o = _stage3(cs, cb)
    o4 = o.reshape(5, B, NFG, KPAD)
    return jnp.moveaxis(o4, 0, 3)[:, :, :MAX_DET, :]


# speedup vs baseline: 1.0686x; 1.0686x over previous
import functools

import jax
import jax.numpy as jnp
from jax import lax
from jax.experimental import pallas as pl
from jax.experimental.pallas import tpu as pltpu
from jax.experimental.pallas import tpu_sc as plsc

TH_CONF = 0.05
TH_IOU = 0.5
MAX_DET = 100
NEG = -1e9
B, N, C = 4, 20000, 21
NFG = C - 1
NROW = B * NFG
KPAD = 128
BIGI = 2**30

K1BASE = 0x3D4C0000 >> 12
NB1 = (0x3F800000 >> 12) - K1BASE + 1
NB1P = ((NB1 + 159) // 160) * 160
NB2 = 4096
NB2P = 4480
NCH = N // 16


def _prep_body(conf_ref, loc_ref, anch_ref, s_ref, bx_ref):
    c = conf_ref[0]
    m = jnp.max(c, axis=0, keepdims=True)
    e = jnp.exp(c - m)
    den = jnp.sum(e, axis=0, keepdims=True)
    s = e[1:, :] / den
    s_ref[0] = jnp.where(s >= TH_CONF, s, NEG)

    l = loc_ref[0]
    a = anch_ref[...]
    acx, acy, aw, ah = a[0:1], a[1:2], a[2:3], a[3:4]
    cx = acx + l[0:1] * 0.1 * aw
    cy = acy + l[1:2] * 0.1 * ah
    w = aw * jnp.exp(l[2:3] * 0.2)
    h = ah * jnp.exp(l[3:4] * 0.2)
    bx_ref[0] = jnp.concatenate(
        [cx - w * 0.5, cy - h * 0.5, cx + w * 0.5, cy + h * 0.5], axis=0)


def _stage1(conf_t, loc_t, anch_t):
    return pl.pallas_call(
        _prep_body,
        grid=(B,),
        in_specs=[
            pl.BlockSpec((1, C, N), lambda b: (b, 0, 0)),
            pl.BlockSpec((1, 4, N), lambda b: (b, 0, 0)),
            pl.BlockSpec((4, N), lambda b: (0, 0)),
        ],
        out_specs=[
            pl.BlockSpec((1, NFG, N), lambda b: (b, 0, 0)),
            pl.BlockSpec((1, 4, N), lambda b: (b, 0, 0)),
        ],
        out_shape=[
            jax.ShapeDtypeStruct((B, NFG, N), jnp.float32),
            jax.ShapeDtypeStruct((B, 4, N), jnp.float32),
        ],
    )(conf_t, loc_t, anch_t)


def _hist_walk(hist_ref, nblk, base_count, lane_iota):
    def cond_f(carry):
        i, found, cum, cut, m = carry
        return jnp.logical_and(i < nblk, found == 0)

    def body_f(carry):
        i, found, cum, cut, m = carry
        for u in range(5):
            blk = nblk - 1 - (i + u)
            h = hist_ref[pl.ds(blk * 16, 16)]
            tot = jnp.sum(h)
            crossing = jnp.logical_and(
                found == 0, (base_count + cum + tot) >= MAX_DET)

            def hit_f(_, blk=blk, h=h, cum=cum):
                rh = lax.rev(h, (0,))
                cs = jnp.cumsum(rh)
                cross = (base_count + cum + cs) >= MAX_DET
                lstar = jnp.min(jnp.where(cross, lane_iota, 16))
                c_at = jnp.sum(jnp.where(lane_iota == lstar, cs, 0))
                h_at = jnp.sum(jnp.where(lane_iota == lstar, rh, 0))
                return (jnp.int32(1), blk * 16 + 15 - lstar,
                        base_count + cum + c_at - h_at)

            def miss_f(_, cut=cut, m=m):
                return (jnp.int32(0), cut, m)

            fnd, cut, m = lax.cond(crossing, hit_f, miss_f, 0)
            found = jnp.maximum(found, fnd)
            cum = cum + jnp.where(found == 0, tot, 0)
        return (i + 5, found, cum, cut, m)

    _, found, cum, cut, m = lax.while_loop(
        cond_f, body_f,
        (jnp.int32(0), jnp.int32(0), jnp.int32(0), jnp.int32(0), jnp.int32(0)))
    return found, cum, cut, m


def _sc_row(r, scores_hbm, boxflat_hbm, so_ref, bo_ref, srow, h1, h2,
            idxb, scb, gidx, grow, cmax, sem):
    lane_iota = lax.iota(jnp.int32, 16)
    pltpu.sync_copy(scores_hbm.at[r], srow)

    zero_v = jnp.zeros((16,), jnp.int32)
    ones_v = jnp.ones((16,), jnp.int32)

    def zero1(i, _):
        for u in range(10):
            h1[pl.ds(i * 160 + u * 16, 16)] = zero_v
        return 0
    lax.fori_loop(0, NB1P // 160, zero1, 0)

    def hpass1(j, _):
        for h in range(2):
            gmax = jnp.full((16,), NEG, jnp.float32)
            for u in range(5):
                v = srow[pl.ds(j * 160 + h * 80 + u * 16, 16)]
                gmax = jnp.maximum(gmax, v)
                msk = v >= TH_CONF
                bits = lax.bitcast_convert_type(v, jnp.int32)
                key = jnp.clip((bits >> 12) - K1BASE, 0, NB1P - 1)
                plsc.addupdate_scatter(h1, [key], ones_v, mask=msk)
            cmax[pl.ds(j * 32 + h * 16, 16)] = gmax
        return 0
    lax.fori_loop(0, NCH // 10, hpass1, 0)

    found1, _, cut1, m1 = _hist_walk(h1, NB1P // 16, jnp.int32(0), lane_iota)

    def zero2(i, _):
        for u in range(8):
            h2[pl.ds(i * 128 + u * 16, 16)] = zero_v
        return 0
    lax.fori_loop(0, NB2P // 128, zero2, 0)

    blo = lax.bitcast_convert_type(
        jnp.broadcast_to((cut1 + K1BASE) << 12, (16,)).astype(jnp.int32),
        jnp.float32)

    def hpass2(j, _):
        for h in range(2):
            cm = cmax[pl.ds(j * 32 + h * 16, 16)]
            hitg = jnp.sum((cm >= blo).astype(jnp.int32)) > 0

            @pl.when(hitg)
            def _():
                for u in range(5):
                    v = srow[pl.ds(j * 160 + h * 80 + u * 16, 16)]
                    bits = lax.bitcast_convert_type(v, jnp.int32)
                    msk = jnp.logical_and(v >= TH_CONF,
                                          ((bits >> 12) - K1BASE) == cut1)
                    key = bits & 0xFFF
                    plsc.addupdate_scatter(h2, [key], ones_v, mask=msk)
        return 0
    lax.fori_loop(0, NCH // 10, hpass2, 0)

    _, _, cut2, m = _hist_walk(h2, NB2P // 16, m1, lane_iota)

    tau_bits = ((cut1 + K1BASE) << 12) | cut2
    tau_vec = lax.bitcast_convert_type(
        jnp.broadcast_to(tau_bits, (16,)).astype(jnp.int32), jnp.float32)
    tau = jnp.where(found1 == 1, tau_vec, jnp.float32(0.0))
    t_tie = jnp.where(found1 == 1, MAX_DET - m, 0)

    def fill(i, _):
        scb[pl.ds(i * 16, 16)] = jnp.full((16,), NEG, jnp.float32)
        idxb[pl.ds(i * 16, 16)] = jnp.zeros((16,), jnp.int32)
        return 0
    lax.fori_loop(0, KPAD // 16, fill, 0)

    def selstep(j, carry):
        off, ties = carry
        cm = cmax[pl.ds(j * 16, 16)]
        hitg = jnp.sum((cm >= tau).astype(jnp.int32)) > 0

        def do_group(c):
            off, ties = c
            for u in range(5):
                v = srow[pl.ds(j * 80 + u * 16, 16)]
                gt = v > tau
                eq = v == tau
                eqc = jnp.cumsum(eq.astype(jnp.int32))
                take_eq = jnp.logical_and(eq, (ties + eqc) <= t_tie)
                sel = jnp.logical_or(gt, take_eq)
                iv = lane_iota + (j * 80 + u * 16)
                plsc.store_compressed(idxb.at[pl.ds(off, 16)], iv, mask=sel)
                plsc.store_compressed(scb.at[pl.ds(off, 16)], v, mask=sel)
                off = off + jnp.sum(sel.astype(jnp.int32))
                ties = ties + jnp.sum(eq.astype(jnp.int32))
            return off, ties

        return lax.cond(hitg, do_group, lambda c: c, (off, ties))

    lax.fori_loop(0, NCH // 5, selstep, (jnp.int32(0), jnp.int32(0)))

    pltpu.sync_copy(scb, so_ref.at[r])

    bq = r // NFG
    for d in range(4):
        def gi(i, _):
            gidx[pl.ds(d * KPAD + i * 16, 16)] = (
                idxb[pl.ds(i * 16, 16)] + (bq * 4 + d) * N)
            return 0
        lax.fori_loop(0, KPAD // 16, gi, 0)
    copies = [
        pltpu.async_copy(boxflat_hbm.at[gidx.at[pl.ds(d * KPAD, KPAD)]],
                         grow.at[pl.ds(d * KPAD, KPAD)], sem)
        for d in range(4)
    ]
    for c in copies:
        c.wait()
    for d in range(4):
        pltpu.sync_copy(grow.at[pl.ds(d * KPAD, KPAD)], bo_ref.at[r, d])


def _sc_body(scores_hbm, boxflat_hbm, so_ref, bo_ref, srow, h1, h2,
             idxb, scb, gidx, grow, cmax, sem):
    wid = lax.axis_index("s") * 2 + lax.axis_index("c")
    for i in range(3):
        r = wid + 32 * i
        @pl.when(r < NROW)
        def _():
            _sc_row(r, scores_hbm, boxflat_hbm, so_ref, bo_ref, srow, h1, h2,
                    idxb, scb, gidx, grow, cmax, sem)


def _stage2(scores, boxflat):
    mesh = plsc.VectorSubcoreMesh(core_axis_name="c", subcore_axis_name="s")
    f = pl.kernel(
        _sc_body,
        mesh=mesh,
        compiler_params=pltpu.CompilerParams(needs_layout_passes=False),
        out_type=[
            jax.ShapeDtypeStruct((NROW, KPAD), jnp.float32),
            jax.ShapeDtypeStruct((NROW, 4, KPAD), jnp.float32),
        ],
        scratch_types=[
            pltpu.VMEM((N,), jnp.float32),
            pltpu.VMEM((NB1P,), jnp.int32),
            pltpu.VMEM((NB2P,), jnp.int32),
            pltpu.VMEM((KPAD,), jnp.int32),
            pltpu.VMEM((KPAD,), jnp.float32),
            pltpu.VMEM((4 * KPAD,), jnp.int32),
            pltpu.VMEM((4 * KPAD,), jnp.float32),
            pltpu.VMEM((4096,), jnp.float32),
            pltpu.SemaphoreType.DMA,
        ],
    )
    return f(scores, boxflat)


def _nms_body(cs_ref, cb_ref, out_ref, s_scr):
    s_scr[...] = cs_ref[...]

    def nms_step(t, _):
        cs = s_scr[...]
        mx = jnp.max(cs, axis=1, keepdims=True)
        iota = lax.broadcasted_iota(jnp.int32, (NROW, KPAD), 1)
        idx = jnp.min(jnp.where(cs == mx, iota, BIGI), axis=1, keepdims=True)
        onehot = iota == idx
        valid = mx > NEG / 2
        cb = cb_ref[...]
        bb = jnp.sum(jnp.where(onehot[None], cb, 0.0), axis=2)
        vrow = valid[:, 0][None, :]
        ob = jnp.where(vrow, bb, 0.0)
        osc = jnp.where(vrow, mx[:, 0][None, :], 0.0)
        val = jnp.concatenate([ob, osc], axis=0)
        slot = iota == t
        out_ref[...] = jnp.where(slot[None], val[:, :, None], out_ref[...])

        bx1, by1 = bb[0][:, None], bb[1][:, None]
        bx2, by2 = bb[2][:, None], bb[3][:, None]
        x1 = jnp.maximum(bx1, cb[0])
        y1 = jnp.maximum(by1, cb[1])
        x2 = jnp.minimum(bx2, cb[2])
        y2 = jnp.minimum(by2, cb[3])
        inter = jnp.maximum(x2 - x1, 0.0) * jnp.maximum(y2 - y1, 0.0)
        area_a = (bx2 - bx1) * (by2 - by1)
        area_b = (cb[2] - cb[0]) * (cb[3] - cb[1])
        iou = inter / (area_a + area_b - inter + 1e-9)
        supp = ((iou > TH_IOU) & valid) | onehot
        s_scr[...] = jnp.where(supp, NEG, cs)
        return 0

    lax.fori_loop(0, MAX_DET, nms_step, 0)


def _stage3(cs, cb):
    return pl.pallas_call(
        _nms_body,
        out_shape=jax.ShapeDtypeStruct((5, NROW, KPAD), jnp.float32),
        scratch_shapes=[pltpu.VMEM((NROW, KPAD), jnp.float32)],
    )(cs, cb)


def kernel(conf, loc, anchors):
    conf_t = jnp.transpose(conf, (0, 2, 1))
    loc_t = jnp.transpose(loc, (0, 2, 1))
    anch_t = jnp.transpose(anchors)

    scores, boxes = _stage1(conf_t, loc_t, anch_t)
    cand_s, cand_b = _stage2(scores.reshape(NROW, N), boxes.reshape(B * 4 * N))
    cs = cand_s
    cb = jnp.transpose(cand_b, (1, 0, 2))
    o = _stage3(cs, cb)
    o4 = o.reshape(5, B, NFG, KPAD)
    return jnp.moveaxis(o4, 0, 3)[:, :, :MAX_DET, :]


# speedup vs baseline: 1.1615x; 1.0870x over previous
import jax
import jax.numpy as jnp
from jax import lax
from jax.experimental import pallas as pl
from jax.experimental.pallas import tpu as pltpu
from jax.experimental.pallas import tpu_sc as plsc

TH_CONF = 0.05
TH_IOU = 0.5
MAX_DET = 100
NEG = -1e9
B, N, C = 4, 20000, 21
NFG = C - 1
NROW = B * NFG
KPAD = 128
BIGI = 2**30

K1BASE = 0x3D4C0000 >> 12
NB1 = (0x3F800000 >> 12) - K1BASE + 1
NB1P = ((NB1 + 15) // 16) * 16
NB2 = 4096
NCH = N // 16


def _prep_body(conf_ref, loc_ref, anch_ref, s_ref, bx_ref):
    c = conf_ref[0]
    m = jnp.max(c, axis=0, keepdims=True)
    e = jnp.exp(c - m)
    den = jnp.sum(e, axis=0, keepdims=True)
    s = e[1:, :] / den
    s_ref[0] = jnp.where(s >= TH_CONF, s, NEG)

    l = loc_ref[0]
    a = anch_ref[...]
    acx, acy, aw, ah = a[0:1], a[1:2], a[2:3], a[3:4]
    cx = acx + l[0:1] * 0.1 * aw
    cy = acy + l[1:2] * 0.1 * ah
    w = aw * jnp.exp(l[2:3] * 0.2)
    h = ah * jnp.exp(l[3:4] * 0.2)
    bx_ref[0] = jnp.concatenate(
        [cx - w * 0.5, cy - h * 0.5, cx + w * 0.5, cy + h * 0.5], axis=0)


def _stage1(conf_t, loc_t, anch_t):
    return pl.pallas_call(
        _prep_body,
        grid=(B,),
        in_specs=[
            pl.BlockSpec((1, C, N), lambda b: (b, 0, 0)),
            pl.BlockSpec((1, 4, N), lambda b: (b, 0, 0)),
            pl.BlockSpec((4, N), lambda b: (0, 0)),
        ],
        out_specs=[
            pl.BlockSpec((1, NFG, N), lambda b: (b, 0, 0)),
            pl.BlockSpec((1, 4, N), lambda b: (b, 0, 0)),
        ],
        out_shape=[
            jax.ShapeDtypeStruct((B, NFG, N), jnp.float32),
            jax.ShapeDtypeStruct((B, 4, N), jnp.float32),
        ],
    )(conf_t, loc_t, anch_t)


def _hist_walk(hist_ref, nblk, base_count, lane_iota):
    def cond_f(carry):
        i, found, cum, cut, m = carry
        return jnp.logical_and(i < nblk, found == 0)

    def body_f(carry):
        i, found, cum, cut, m = carry
        blk = nblk - 1 - i
        h = hist_ref[pl.ds(blk * 16, 16)]
        tot = jnp.sum(h)
        crossing = (base_count + cum + tot) >= MAX_DET

        def hit_f(_):
            rh = lax.rev(h, (0,))
            cs = jnp.cumsum(rh)
            cross = (base_count + cum + cs) >= MAX_DET
            lstar = jnp.min(jnp.where(cross, lane_iota, 16))
            c_at = jnp.sum(jnp.where(lane_iota == lstar, cs, 0))
            h_at = jnp.sum(jnp.where(lane_iota == lstar, rh, 0))
            return (jnp.int32(1), blk * 16 + 15 - lstar,
                    base_count + cum + c_at - h_at)

        def miss_f(_):
            return (jnp.int32(0), cut, m)

        found, cut, m = lax.cond(crossing, hit_f, miss_f, 0)
        return (i + 1, found, cum + tot, cut, m)

    _, found, cum, cut, m = lax.while_loop(
        cond_f, body_f,
        (jnp.int32(0), jnp.int32(0), jnp.int32(0), jnp.int32(0), jnp.int32(0)))
    return found, cum, cut, m


def _sc_row(r, scores_hbm, boxflat_hbm, so_ref, bo_ref, srow, h1, h2,
            idxb, scb, gidx, grow, cmax, sem):
    lane_iota = lax.iota(jnp.int32, 16)
    pltpu.sync_copy(scores_hbm.at[r], srow)

    zero_v = jnp.zeros((16,), jnp.int32)
    ones_v = jnp.ones((16,), jnp.int32)

    def zero1(i, _):
        for u in range(5):
            h1[pl.ds(i * 80 + u * 16, 16)] = zero_v
        return 0
    lax.fori_loop(0, NB1P // 80, zero1, 0)

    def hpass1(j, _):
        gmax = jnp.full((16,), NEG, jnp.float32)
        for u in range(5):
            v = srow[pl.ds(j * 80 + u * 16, 16)]
            gmax = jnp.maximum(gmax, v)
            msk = v >= TH_CONF
            bits = lax.bitcast_convert_type(v, jnp.int32)
            key = jnp.clip((bits >> 12) - K1BASE, 0, NB1P - 1)
            plsc.addupdate_scatter(h1, [key], ones_v, mask=msk)
        cmax[pl.ds(j * 16, 16)] = gmax
        return 0
    lax.fori_loop(0, NCH // 5, hpass1, 0)

    found1, _, cut1, m1 = _hist_walk(h1, NB1P // 16, jnp.int32(0), lane_iota)

    def zero2(i, _):
        for u in range(4):
            h2[pl.ds(i * 64 + u * 16, 16)] = zero_v
        return 0
    lax.fori_loop(0, NB2 // 64, zero2, 0)

    blo = lax.bitcast_convert_type(
        jnp.broadcast_to((cut1 + K1BASE) << 12, (16,)).astype(jnp.int32),
        jnp.float32)

    def hpass2(j, _):
        cm = cmax[pl.ds(j * 16, 16)]
        hitg = jnp.sum((cm >= blo).astype(jnp.int32)) > 0

        @pl.when(hitg)
        def _():
            for u in range(5):
                v = srow[pl.ds(j * 80 + u * 16, 16)]
                bits = lax.bitcast_convert_type(v, jnp.int32)
                msk = jnp.logical_and(v >= TH_CONF,
                                      ((bits >> 12) - K1BASE) == cut1)
                key = bits & 0xFFF
                plsc.addupdate_scatter(h2, [key], ones_v, mask=msk)
        return 0
    lax.fori_loop(0, NCH // 5, hpass2, 0)

    _, _, cut2, m = _hist_walk(h2, NB2 // 16, m1, lane_iota)

    tau_bits = ((cut1 + K1BASE) << 12) | cut2
    tau_vec = lax.bitcast_convert_type(
        jnp.broadcast_to(tau_bits, (16,)).astype(jnp.int32), jnp.float32)
    tau = jnp.where(found1 == 1, tau_vec, jnp.float32(0.0))
    t_tie = jnp.where(found1 == 1, MAX_DET - m, 0)

    def fill(i, _):
        scb[pl.ds(i * 16, 16)] = jnp.full((16,), NEG, jnp.float32)
        idxb[pl.ds(i * 16, 16)] = jnp.zeros((16,), jnp.int32)
        return 0
    lax.fori_loop(0, KPAD // 16, fill, 0)

    def selstep(j, carry):
        off, ties = carry
        cm = cmax[pl.ds(j * 16, 16)]
        hitg = jnp.sum((cm >= tau).astype(jnp.int32)) > 0

        def do_group(c):
            off, ties = c
            for u in range(5):
                v = srow[pl.ds(j * 80 + u * 16, 16)]
                gt = v > tau
                eq = v == tau
                eqc = jnp.cumsum(eq.astype(jnp.int32))
                take_eq = jnp.logical_and(eq, (ties + eqc) <= t_tie)
                sel = jnp.logical_or(gt, take_eq)
                iv = lane_iota + (j * 80 + u * 16)
                plsc.store_compressed(idxb.at[pl.ds(off, 16)], iv, mask=sel)
                plsc.store_compressed(scb.at[pl.ds(off, 16)], v, mask=sel)
                off = off + jnp.sum(sel.astype(jnp.int32))
                ties = ties + jnp.sum(eq.astype(jnp.int32))
            return off, ties

        return lax.cond(hitg, do_group, lambda c: c, (off, ties))

    lax.fori_loop(0, NCH // 5, selstep, (jnp.int32(0), jnp.int32(0)))

    pltpu.sync_copy(scb, so_ref.at[r])

    bq = r // NFG
    for d in range(4):
        def gi(i, _):
            gidx[pl.ds(d * KPAD + i * 16, 16)] = (
                idxb[pl.ds(i * 16, 16)] + (bq * 4 + d) * N)
            return 0
        lax.fori_loop(0, KPAD // 16, gi, 0)
    copies = [
        pltpu.async_copy(boxflat_hbm.at[gidx.at[pl.ds(d * KPAD, KPAD)]],
                         grow.at[pl.ds(d * KPAD, KPAD)], sem)
        for d in range(4)
    ]
    for c in copies:
        c.wait()
    for d in range(4):
        pltpu.sync_copy(grow.at[pl.ds(d * KPAD, KPAD)], bo_ref.at[r, d])


def _sc_body(scores_hbm, boxflat_hbm, so_ref, bo_ref, srow, h1, h2,
             idxb, scb, gidx, grow, cmax, sem):
    wid = lax.axis_index("s") * 2 + lax.axis_index("c")
    for i in range(3):
        r = wid + 32 * i
        @pl.when(r < NROW)
        def _():
            _sc_row(r, scores_hbm, boxflat_hbm, so_ref, bo_ref, srow, h1, h2,
                    idxb, scb, gidx, grow, cmax, sem)


def _stage2(scores, boxflat):
    mesh = plsc.VectorSubcoreMesh(core_axis_name="c", subcore_axis_name="s")
    f = pl.kernel(
        _sc_body,
        mesh=mesh,
        compiler_params=pltpu.CompilerParams(needs_layout_passes=False),
        out_type=[
            jax.ShapeDtypeStruct((NROW, KPAD), jnp.float32),
            jax.ShapeDtypeStruct((NROW, 4, KPAD), jnp.float32),
        ],
        scratch_types=[
            pltpu.VMEM((N,), jnp.float32),
            pltpu.VMEM((NB1P,), jnp.int32),
            pltpu.VMEM((NB2,), jnp.int32),
            pltpu.VMEM((KPAD,), jnp.int32),
            pltpu.VMEM((KPAD,), jnp.float32),
            pltpu.VMEM((4 * KPAD,), jnp.int32),
            pltpu.VMEM((4 * KPAD,), jnp.float32),
            pltpu.VMEM((4096,), jnp.float32),
            pltpu.SemaphoreType.DMA,
        ],
    )
    return f(scores, boxflat)


def _nms_body(cs_ref, cb_ref, out_ref, s_scr):
    s_scr[...] = cs_ref[...]

    def nms_step(t, _):
        cs = s_scr[...]
        mx = jnp.max(cs, axis=1, keepdims=True)
        iota = lax.broadcasted_iota(jnp.int32, (NROW, KPAD), 1)
        idx = jnp.min(jnp.where(cs == mx, iota, BIGI), axis=1, keepdims=True)
        onehot = iota == idx
        valid = mx > NEG / 2
        cb = cb_ref[...]
        bb = jnp.sum(jnp.where(onehot[None], cb, 0.0), axis=2)
        vrow = valid[:, 0][None, :]
        ob = jnp.where(vrow, bb, 0.0)
        osc = jnp.where(vrow, mx[:, 0][None, :], 0.0)
        val = jnp.concatenate([ob, osc], axis=0)
        slot = iota == t
        out_ref[...] = jnp.where(slot[None], val[:, :, None], out_ref[...])

        bx1, by1 = bb[0][:, None], bb[1][:, None]
        bx2, by2 = bb[2][:, None], bb[3][:, None]
        x1 = jnp.maximum(bx1, cb[0])
        y1 = jnp.maximum(by1, cb[1])
        x2 = jnp.minimum(bx2, cb[2])
        y2 = jnp.minimum(by2, cb[3])
        inter = jnp.maximum(x2 - x1, 0.0) * jnp.maximum(y2 - y1, 0.0)
        area_a = (bx2 - bx1) * (by2 - by1)
        area_b = (cb[2] - cb[0]) * (cb[3] - cb[1])
        iou = inter / (area_a + area_b - inter + 1e-9)
        supp = ((iou > TH_IOU) & valid) | onehot
        s_scr[...] = jnp.where(supp, NEG, cs)
        return 0

    lax.fori_loop(0, MAX_DET, nms_step, 0)


def _stage3(cs, cb):
    return pl.pallas_call(
        _nms_body,
        out_shape=jax.ShapeDtypeStruct((5, NROW, KPAD), jnp.float32),
        scratch_shapes=[pltpu.VMEM((NROW, KPAD), jnp.float32)],
    )(cs, cb)


def kernel(conf, loc, anchors):
    conf_t = jnp.transpose(conf, (0, 2, 1))
    loc_t = jnp.transpose(loc, (0, 2, 1))
    anch_t = jnp.transpose(anchors)

    scores, boxes = _stage1(conf_t, loc_t, anch_t)
    cand_s, cand_b = _stage2(scores.reshape(NROW, N), boxes.reshape(B * 4 * N))
    cs = cand_s
    cb = jnp.transpose(cand_b, (1, 0, 2))
    o = _stage3(cs, cb)
    o4 = o.reshape(5, B, NFG, KPAD)
    return jnp.moveaxis(o4, 0, 3)[:, :, :MAX_DET, :]
